# baseline (speedup 1.0000x reference)
import functools

import numpy as np
import jax
import jax.numpy as jnp
from jax import lax
from jax.experimental import pallas as pl
from jax.experimental.pallas import tpu as pltpu
from jax.experimental.pallas import tpu_sc as plsc

_T = 4
_D = 1024
_N = 8192
_NW = 32
_G = 16
_NT = 4
_CH = _D // 16
_TPW = _N // _NW
_NG = _TPW // _G

_GDN = lax.GatherDimensionNumbers(
    offset_dims=(), collapsed_slice_dims=(0,), start_index_map=(0,))


def _lane_perm(v, idx):
    return lax.gather(v, idx, _GDN, (1,),
                      mode=lax.GatherScatterMode.PROMISE_IN_BOUNDS)


def _round_bf16(v):
    y = lax.bitcast_convert_type(v, jnp.int32)
    r = (y + 0x7FFF + ((y >> 16) & 1)) & jnp.int32(-65536)
    return lax.bitcast_convert_type(r, jnp.float32)


def _sc_body(x0, x1, x2, x3, w, params, out, xb, ob, wb, pb):
    wid = lax.axis_index("c") * 16 + lax.axis_index("s")
    pltpu.sync_copy(w, wb)
    pltpu.sync_copy(params, pb)
    lanes = lax.iota(jnp.int32, 16)
    perms = {sh: lax.reshape(lanes ^ sh, (16, 1)) for sh in (8, 4, 2, 1)}
    pv = pb[...]
    av = jnp.full((16,), 1.0, jnp.float32) * pv[8]
    a = 1.0 / (1.0 + jnp.exp(-av))
    one_m_a = 1.0 - a
    f32 = jnp.float32
    xsrc = (x0, x1, x2, x3)

    def group(g, carry):
        base = (wid * _TPW + g * _G) * _D
        for t in range(_T):
            pltpu.sync_copy(xsrc[t].at[pl.ds(base, _G * _D)],
                            xb.at[pl.ds(t * _G * _D, _G * _D)])

        L = [jnp.zeros((16,), f32) for _ in range(8)]
        for blk in range(_G // _NT):
            def jbody(j, accs):
                accs = list(accs)
                for t in range(_T):
                    wo = (t * _CH) * 8 * 16
                    wv = [wb[pl.ds(wo + j * 128 + k * 16, 16)]
                          for k in range(8)]
                    for n in range(_NT):
                        xo = (t * _G + blk * _NT + n) * _D
                        xv = _round_bf16(xb[pl.ds(xo + j * 16, 16)])
                        for k in range(8):
                            accs[n * 8 + k] = accs[n * 8 + k] + xv * wv[k]
                return tuple(accs)

            accs = lax.fori_loop(
                0, _CH, jbody,
                tuple(jnp.zeros((16,), f32) for _ in range(_NT * 8)))
            for n in range(_NT):
                tok = blk * _NT + n
                for k in range(8):
                    s = accs[n * 8 + k]
                    for sh in (8, 4, 2, 1):
                        s = s + _lane_perm(s, perms[sh])
                    L[k] = jnp.where(lanes == tok, s, L[k])

        lt = [L[k] + pv[k] for k in range(4)]
        ls = [L[4 + k] + pv[4 + k] for k in range(4)]
        v1 = jnp.maximum(jnp.maximum(lt[0], lt[1]), jnp.maximum(lt[2], lt[3]))
        i1 = jnp.where(lt[0] >= v1, 0,
                       jnp.where(lt[1] >= v1, 1,
                                 jnp.where(lt[2] >= v1, 2, 3)))
        neg = jnp.float32(-3.0e38)
        l2 = [jnp.where(i1 == k, neg, lt[k]) for k in range(4)]
        v2 = jnp.maximum(jnp.maximum(l2[0], l2[1]), jnp.maximum(l2[2], l2[3]))
        i2 = jnp.where(l2[0] >= v2, 0,
                       jnp.where(l2[1] >= v2, 1,
                                 jnp.where(l2[2] >= v2, 2, 3)))
        e2 = jnp.exp(v2 - v1)
        p1 = 1.0 / (1.0 + e2)
        p2 = 1.0 - p1
        ms = jnp.maximum(jnp.maximum(ls[0], ls[1]), jnp.maximum(ls[2], ls[3]))
        es = [jnp.exp(ls[k] - ms) for k in range(4)]
        den = es[0] + es[1] + es[2] + es[3]
        wts = []
        for k in range(4):
            tw = (jnp.where(i1 == k, p1, 0.0) + jnp.where(i2 == k, p2, 0.0))
            wts.append(a * tw + one_m_a * (es[k] / den))

        for n in range(_G):
            s0 = wts[0][n]
            s1 = wts[1][n]
            s2 = wts[2][n]
            s3 = wts[3][n]

            def cbody(c, carry2):
                o = c * 16
                ob[pl.ds(n * _D + o, 16)] = (
                    xb[pl.ds((0 * _G + n) * _D + o, 16)] * s0
                    + xb[pl.ds((1 * _G + n) * _D + o, 16)] * s1
                    + xb[pl.ds((2 * _G + n) * _D + o, 16)] * s2
                    + xb[pl.ds((3 * _G + n) * _D + o, 16)] * s3)
                return carry2

            lax.fori_loop(0, _CH, cbody, 0)

        pltpu.sync_copy(ob, out.at[pl.ds(base, _G * _D)])
        return carry

    lax.fori_loop(0, _NG, group, 0)


def _build_sc_call():
    mesh = plsc.VectorSubcoreMesh(core_axis_name="c", subcore_axis_name="s")
    return pl.kernel(
        _sc_body,
        mesh=mesh,
        out_type=jax.ShapeDtypeStruct((_N * _D,), jnp.float32),
        scratch_types=[
            pltpu.VMEM((_T * _G * _D,), jnp.float32),
            pltpu.VMEM((_G * _D,), jnp.float32),
            pltpu.VMEM((_T * _CH * 8 * 16,), jnp.float32),
            pltpu.VMEM((16,), jnp.float32),
        ],
    )


def kernel(mod0, mod1, mod2, mod3, W_top, b_top, W_soft, b_soft, alpha):
    B, S, D = mod0.shape
    N = B * S
    xs = [m.reshape(N * D) for m in (mod0, mod1, mod2, mod3)]

    wt = W_top.reshape(_T, D, _T).transpose(2, 1, 0)
    ws = W_soft.reshape(_T, D, _T).transpose(2, 1, 0)
    w = jnp.concatenate([wt, ws], axis=-1)
    w = w.astype(jnp.bfloat16).astype(jnp.float32)
    w = w.reshape(_T, _CH, 16, 8).transpose(0, 1, 3, 2).reshape(-1)
    params = jnp.concatenate(
        [b_top, b_soft, alpha, jnp.zeros((7,), jnp.float32)])

    out = _build_sc_call()(xs[0], xs[1], xs[2], xs[3], w, params)
    return out.reshape(B, S, D)


# speedup vs baseline: 1.2146x; 1.2146x over previous
import functools

import numpy as np
import jax
import jax.numpy as jnp
from jax import lax
from jax.experimental import pallas as pl
from jax.experimental.pallas import tpu as pltpu
from jax.experimental.pallas import tpu_sc as plsc

_T = 4
_D = 1024
_N = 8192
_NW = 32
_G = 8
_NT = 4
_CH = _D // 16
_TPW = _N // _NW
_NG = _TPW // _G
_GB = _G * _D

_GDN = lax.GatherDimensionNumbers(
    offset_dims=(), collapsed_slice_dims=(0,), start_index_map=(0,))


def _lane_perm(v, idx):
    return lax.gather(v, idx, _GDN, (1,),
                      mode=lax.GatherScatterMode.PROMISE_IN_BOUNDS)


def _round_bf16(v):
    y = lax.bitcast_convert_type(v, jnp.int32)
    r = (y + 0x7FFF + ((y >> 16) & 1)) & jnp.int32(-65536)
    return lax.bitcast_convert_type(r, jnp.float32)


def _round_pair(a, b):
    return _round_bf16(a), _round_bf16(b)


def _sc_body(x0, x1, x2, x3, w, params, out, xb, ob, wb, pb,
             si0, si1, so0, so1):
    wid = lax.axis_index("c") * 16 + lax.axis_index("s")
    pltpu.sync_copy(w, wb)
    pltpu.sync_copy(params, pb)
    lanes = lax.iota(jnp.int32, 16)
    perms = {sh: lax.reshape(lanes ^ sh, (16, 1)) for sh in (8, 4, 2, 1)}
    pv = pb[...]
    av = jnp.full((16,), 1.0, jnp.float32) * pv[8]
    a = 1.0 / (1.0 + jnp.exp(-av))
    one_m_a = 1.0 - a
    f32 = jnp.float32
    xsrc = (x0, x1, x2, x3)
    tok0 = wid * _TPW

    def start_in(g, par, sem):
        base = (tok0 + g * _G) * _D
        for t in range(_T):
            pltpu.async_copy(xsrc[t].at[pl.ds(base, _GB)],
                             xb.at[pl.ds((par * _T + t) * _GB, _GB)], sem)

    def drain_in(par, sem):
        for t in range(_T):
            pltpu.make_async_copy(
                xsrc[t].at[pl.ds(0, _GB)],
                xb.at[pl.ds((par * _T + t) * _GB, _GB)], sem).wait()

    start_in(0, 0, si0)
    start_in(1, 1, si1)

    def run_group(g, par, si, so):
        drain_in(par, si)

        L = [jnp.zeros((16,), f32) for _ in range(8)]
        for blk in range(_G // _NT):
            def jbody(jj, accs):
                accs = list(accs)
                for t in range(_T):
                    wo = t * _CH * 128
                    wA = [wb[pl.ds(wo + (2 * jj) * 128 + k * 16, 16)]
                          for k in range(8)]
                    wB = [wb[pl.ds(wo + (2 * jj + 1) * 128 + k * 16, 16)]
                          for k in range(8)]
                    for n in range(_NT):
                        xo = ((par * _T + t) * _G + blk * _NT + n) * _D
                        r0, r1 = _round_pair(
                            xb[pl.ds(xo + (2 * jj) * 16, 16)],
                            xb[pl.ds(xo + (2 * jj + 1) * 16, 16)])
                        for k in range(8):
                            accs[n * 8 + k] = (accs[n * 8 + k]
                                               + r0 * wA[k] + r1 * wB[k])
                return tuple(accs)

            accs = lax.fori_loop(
                0, _CH // 2, jbody,
                tuple(jnp.zeros((16,), f32) for _ in range(_NT * 8)))
            for n in range(_NT):
                tok = blk * _NT + n
                for k in range(8):
                    s = accs[n * 8 + k]
                    for sh in (8, 4, 2, 1):
                        s = s + _lane_perm(s, perms[sh])
                    L[k] = jnp.where(lanes == tok, s, L[k])

        lt = [L[k] + pv[k] for k in range(4)]
        ls = [L[4 + k] + pv[4 + k] for k in range(4)]
        v1 = jnp.maximum(jnp.maximum(lt[0], lt[1]), jnp.maximum(lt[2], lt[3]))
        i1 = jnp.where(lt[0] >= v1, 0,
                       jnp.where(lt[1] >= v1, 1,
                                 jnp.where(lt[2] >= v1, 2, 3)))
        neg = jnp.float32(-3.0e38)
        l2 = [jnp.where(i1 == k, neg, lt[k]) for k in range(4)]
        v2 = jnp.maximum(jnp.maximum(l2[0], l2[1]), jnp.maximum(l2[2], l2[3]))
        i2 = jnp.where(l2[0] >= v2, 0,
                       jnp.where(l2[1] >= v2, 1,
                                 jnp.where(l2[2] >= v2, 2, 3)))
        e2 = jnp.exp(v2 - v1)
        p1 = 1.0 / (1.0 + e2)
        p2 = 1.0 - p1
        ms = jnp.maximum(jnp.maximum(ls[0], ls[1]), jnp.maximum(ls[2], ls[3]))
        es = [jnp.exp(ls[k] - ms) for k in range(4)]
        den = es[0] + es[1] + es[2] + es[3]
        wts = []
        for k in range(4):
            tw = (jnp.where(i1 == k, p1, 0.0) + jnp.where(i2 == k, p2, 0.0))
            wts.append(a * tw + one_m_a * (es[k] / den))

        obase = par * _GB
        for n in range(_G):
            s0 = wts[0][n]
            s1 = wts[1][n]
            s2 = wts[2][n]
            s3 = wts[3][n]
            xo0 = ((par * _T + 0) * _G + n) * _D
            xo1 = ((par * _T + 1) * _G + n) * _D
            xo2 = ((par * _T + 2) * _G + n) * _D
            xo3 = ((par * _T + 3) * _G + n) * _D

            def cbody(cc, carry2):
                for u in range(4):
                    o = (cc * 4 + u) * 16
                    ob[pl.ds(obase + n * _D + o, 16)] = (
                        xb[pl.ds(xo0 + o, 16)] * s0
                        + xb[pl.ds(xo1 + o, 16)] * s1
                        + xb[pl.ds(xo2 + o, 16)] * s2
                        + xb[pl.ds(xo3 + o, 16)] * s3)
                return carry2

            lax.fori_loop(0, _CH // 4, cbody, 0)

        base = (tok0 + g * _G) * _D
        pltpu.async_copy(ob.at[pl.ds(obase, _GB)],
                         out.at[pl.ds(base, _GB)], so)

    def pair(i, carry):
        for par, si, so in ((0, si0, so0), (1, si1, so1)):
            g = 2 * i + par

            @pl.when(i >= 1)
            def _():
                pltpu.make_async_copy(ob.at[pl.ds(par * _GB, _GB)],
                                      out.at[pl.ds(0, _GB)], so).wait()

            run_group(g, par, si, so)

            @pl.when(i < (_NG // 2) - 1)
            def _():
                start_in(g + 2, par, si)
        return carry

    lax.fori_loop(0, _NG // 2, pair, 0)

    for par, so in ((0, so0), (1, so1)):
        pltpu.make_async_copy(ob.at[pl.ds(par * _GB, _GB)],
                              out.at[pl.ds(0, _GB)], so).wait()


def _build_sc_call():
    mesh = plsc.VectorSubcoreMesh(core_axis_name="c", subcore_axis_name="s")
    return pl.kernel(
        _sc_body,
        mesh=mesh,
        out_type=jax.ShapeDtypeStruct((_N * _D,), jnp.float32),
        scratch_types=[
            pltpu.VMEM((2 * _T * _GB,), jnp.float32),
            pltpu.VMEM((2 * _GB,), jnp.float32),
            pltpu.VMEM((_T * _CH * 8 * 16,), jnp.float32),
            pltpu.VMEM((16,), jnp.float32),
            pltpu.SemaphoreType.DMA,
            pltpu.SemaphoreType.DMA,
            pltpu.SemaphoreType.DMA,
            pltpu.SemaphoreType.DMA,
        ],
    )


def kernel(mod0, mod1, mod2, mod3, W_top, b_top, W_soft, b_soft, alpha):
    B, S, D = mod0.shape
    N = B * S
    xs = [m.reshape(N * D) for m in (mod0, mod1, mod2, mod3)]

    wt = W_top.reshape(_T, D, _T).transpose(2, 1, 0)
    ws = W_soft.reshape(_T, D, _T).transpose(2, 1, 0)
    w = jnp.concatenate([wt, ws], axis=-1)
    w = w.astype(jnp.bfloat16).astype(jnp.float32)
    w = w.reshape(_T, _CH, 16, 8).transpose(0, 1, 3, 2).reshape(-1)
    params = jnp.concatenate(
        [b_top, b_soft, alpha, jnp.zeros((7,), jnp.float32)])

    out = _build_sc_call()(xs[0], xs[1], xs[2], xs[3], w, params)
    return out.reshape(B, S, D)


# speedup vs baseline: 1.4959x; 1.2316x over previous
import functools

import numpy as np
import jax
import jax.numpy as jnp
from jax import lax
from jax.experimental import pallas as pl
from jax.experimental.pallas import tpu as pltpu
from jax.experimental.pallas import tpu_sc as plsc

_T = 4
_D = 1024
_N = 8192
_NW = 32
_G = 8
_NT = 2
_CH = _D // 16
_TPW = _N // _NW
_NG = _TPW // _G
_GB = _G * _D

_GDN = lax.GatherDimensionNumbers(
    offset_dims=(), collapsed_slice_dims=(0,), start_index_map=(0,))


def _lane_perm(v, idx):
    return lax.gather(v, idx, _GDN, (1,),
                      mode=lax.GatherScatterMode.PROMISE_IN_BOUNDS)


def _round_bf16(v):
    y = lax.bitcast_convert_type(v, jnp.int32)
    r = (y + 0x7FFF + ((y >> 16) & 1)) & jnp.int32(-65536)
    return lax.bitcast_convert_type(r, jnp.float32)


def _round_pair(a, b):
    return _round_bf16(a), _round_bf16(b)


def _sc_body(x0, x1, x2, x3, w, params, out, xb, ob, wb, pb,
             si0, si1, so0, so1):
    wid = lax.axis_index("c") * 16 + lax.axis_index("s")
    pltpu.sync_copy(w, wb)
    pltpu.sync_copy(params, pb)
    lanes = lax.iota(jnp.int32, 16)
    perms = {sh: lax.reshape(lanes ^ sh, (16, 1)) for sh in (8, 4, 2, 1)}
    pv = pb[...]
    av = jnp.full((16,), 1.0, jnp.float32) * pv[8]
    a = 1.0 / (1.0 + jnp.exp(-av))
    one_m_a = 1.0 - a
    f32 = jnp.float32
    xsrc = (x0, x1, x2, x3)
    tok0 = wid * _TPW

    def start_in(g, par, sem):
        base = (tok0 + g * _G) * _D
        for t in range(_T):
            pltpu.async_copy(xsrc[t].at[pl.ds(base, _GB)],
                             xb.at[pl.ds((par * _T + t) * _GB, _GB)], sem)

    def drain_in(par, sem):
        for t in range(_T):
            pltpu.make_async_copy(
                xsrc[t].at[pl.ds(0, _GB)],
                xb.at[pl.ds((par * _T + t) * _GB, _GB)], sem).wait()

    start_in(0, 0, si0)
    start_in(1, 1, si1)

    def run_group(g, par, si, so):
        drain_in(par, si)

        L = [jnp.zeros((16,), f32) for _ in range(8)]
        for blk in range(_G // _NT):
            def jbody(jj, accs):
                accs = list(accs)
                for t in range(_T):
                    wo = t * _CH * 128
                    wA = [wb[pl.ds(wo + (2 * jj) * 128 + k * 16, 16)]
                          for k in range(8)]
                    wB = [wb[pl.ds(wo + (2 * jj + 1) * 128 + k * 16, 16)]
                          for k in range(8)]
                    for n in range(_NT):
                        xo = ((par * _T + t) * _G + blk * _NT + n) * _D
                        r0, r1 = _round_pair(
                            xb[pl.ds(xo + (2 * jj) * 16, 16)],
                            xb[pl.ds(xo + (2 * jj + 1) * 16, 16)])
                        for k in range(8):
                            accs[n * 8 + k] = (accs[n * 8 + k]
                                               + r0 * wA[k] + r1 * wB[k])
                return tuple(accs)

            accs = lax.fori_loop(
                0, _CH // 2, jbody,
                tuple(jnp.zeros((16,), f32) for _ in range(_NT * 8)))
            for n in range(_NT):
                tok = blk * _NT + n
                for k in range(8):
                    s = accs[n * 8 + k]
                    for sh in (8, 4, 2, 1):
                        s = s + _lane_perm(s, perms[sh])
                    L[k] = jnp.where(lanes == tok, s, L[k])

        lt = [L[k] + pv[k] for k in range(4)]
        ls = [L[4 + k] + pv[4 + k] for k in range(4)]
        v1 = jnp.maximum(jnp.maximum(lt[0], lt[1]), jnp.maximum(lt[2], lt[3]))
        i1 = jnp.where(lt[0] >= v1, 0,
                       jnp.where(lt[1] >= v1, 1,
                                 jnp.where(lt[2] >= v1, 2, 3)))
        neg = jnp.float32(-3.0e38)
        l2 = [jnp.where(i1 == k, neg, lt[k]) for k in range(4)]
        v2 = jnp.maximum(jnp.maximum(l2[0], l2[1]), jnp.maximum(l2[2], l2[3]))
        i2 = jnp.where(l2[0] >= v2, 0,
                       jnp.where(l2[1] >= v2, 1,
                                 jnp.where(l2[2] >= v2, 2, 3)))
        e2 = jnp.exp(v2 - v1)
        p1 = 1.0 / (1.0 + e2)
        p2 = 1.0 - p1
        ms = jnp.maximum(jnp.maximum(ls[0], ls[1]), jnp.maximum(ls[2], ls[3]))
        es = [jnp.exp(ls[k] - ms) for k in range(4)]
        den = es[0] + es[1] + es[2] + es[3]
        wts = []
        for k in range(4):
            tw = (jnp.where(i1 == k, p1, 0.0) + jnp.where(i2 == k, p2, 0.0))
            wts.append(a * tw + one_m_a * (es[k] / den))

        obase = par * _GB
        for n in range(_G):
            s0 = wts[0][n]
            s1 = wts[1][n]
            s2 = wts[2][n]
            s3 = wts[3][n]
            xo0 = ((par * _T + 0) * _G + n) * _D
            xo1 = ((par * _T + 1) * _G + n) * _D
            xo2 = ((par * _T + 2) * _G + n) * _D
            xo3 = ((par * _T + 3) * _G + n) * _D

            def cbody(cc, carry2):
                for u in range(4):
                    o = (cc * 4 + u) * 16
                    ob[pl.ds(obase + n * _D + o, 16)] = (
                        xb[pl.ds(xo0 + o, 16)] * s0
                        + xb[pl.ds(xo1 + o, 16)] * s1
                        + xb[pl.ds(xo2 + o, 16)] * s2
                        + xb[pl.ds(xo3 + o, 16)] * s3)
                return carry2

            lax.fori_loop(0, _CH // 4, cbody, 0)

        base = (tok0 + g * _G) * _D
        pltpu.async_copy(ob.at[pl.ds(obase, _GB)],
                         out.at[pl.ds(base, _GB)], so)

    def pair(i, carry):
        for par, si, so in ((0, si0, so0), (1, si1, so1)):
            g = 2 * i + par

            @pl.when(i >= 1)
            def _():
                pltpu.make_async_copy(ob.at[pl.ds(par * _GB, _GB)],
                                      out.at[pl.ds(0, _GB)], so).wait()

            run_group(g, par, si, so)

            @pl.when(i < (_NG // 2) - 1)
            def _():
                start_in(g + 2, par, si)
        return carry

    lax.fori_loop(0, _NG // 2, pair, 0)

    for par, so in ((0, so0), (1, so1)):
        pltpu.make_async_copy(ob.at[pl.ds(par * _GB, _GB)],
                              out.at[pl.ds(0, _GB)], so).wait()


def _build_sc_call():
    mesh = plsc.VectorSubcoreMesh(core_axis_name="c", subcore_axis_name="s")
    return pl.kernel(
        _sc_body,
        mesh=mesh,
        out_type=jax.ShapeDtypeStruct((_N * _D,), jnp.float32),
        scratch_types=[
            pltpu.VMEM((2 * _T * _GB,), jnp.float32),
            pltpu.VMEM((2 * _GB,), jnp.float32),
            pltpu.VMEM((_T * _CH * 8 * 16,), jnp.float32),
            pltpu.VMEM((16,), jnp.float32),
            pltpu.SemaphoreType.DMA,
            pltpu.SemaphoreType.DMA,
            pltpu.SemaphoreType.DMA,
            pltpu.SemaphoreType.DMA,
        ],
    )


def kernel(mod0, mod1, mod2, mod3, W_top, b_top, W_soft, b_soft, alpha):
    B, S, D = mod0.shape
    N = B * S
    xs = [m.reshape(N * D) for m in (mod0, mod1, mod2, mod3)]

    wt = W_top.reshape(_T, D, _T).transpose(2, 1, 0)
    ws = W_soft.reshape(_T, D, _T).transpose(2, 1, 0)
    w = jnp.concatenate([wt, ws], axis=-1)
    w = w.astype(jnp.bfloat16).astype(jnp.float32)
    w = w.reshape(_T, _CH, 16, 8).transpose(0, 1, 3, 2).reshape(-1)
    params = jnp.concatenate(
        [b_top, b_soft, alpha, jnp.zeros((7,), jnp.float32)])

    out = _build_sc_call()(xs[0], xs[1], xs[2], xs[3], w, params)
    return out.reshape(B, S, D)


# speedup vs baseline: 3.5057x; 2.3435x over previous
import functools

import numpy as np
import jax
import jax.numpy as jnp
from jax import lax
from jax.experimental import pallas as pl
from jax.experimental.pallas import tpu as pltpu
from jax.experimental.pallas import tpu_sc as plsc

_T = 4
_D = 1024
_N = 8192
_NSC = 512
_NTC = _N - _NSC
_TB = 512

_NW = 32
_G = 8
_NT = 2
_CH = _D // 16
_TPW = _NSC // _NW
_NG = _TPW // _G
_GB = _G * _D


def _tc_body(x0, x1, x2, x3, w, bias, al, out):
    f32 = jnp.float32
    logits = (
        jax.lax.dot_general(x0[...], w[0], (((1,), (0,)), ((), ())),
                            preferred_element_type=f32)
        + jax.lax.dot_general(x1[...], w[1], (((1,), (0,)), ((), ())),
                              preferred_element_type=f32)
        + jax.lax.dot_general(x2[...], w[2], (((1,), (0,)), ((), ())),
                              preferred_element_type=f32)
        + jax.lax.dot_general(x3[...], w[3], (((1,), (0,)), ((), ())),
                              preferred_element_type=f32)
    ) + bias[0, :]
    lt = logits[:, :_T]
    ls = logits[:, _T:]

    col = jax.lax.broadcasted_iota(jnp.int32, (_TB, _T), 1)
    v1 = jnp.max(lt, axis=-1, keepdims=True)
    i1 = jnp.min(jnp.where(lt >= v1, col, _T), axis=-1, keepdims=True)
    m1 = col == i1
    lt2 = jnp.where(m1, -jnp.inf, lt)
    v2 = jnp.max(lt2, axis=-1, keepdims=True)
    i2 = jnp.min(jnp.where(lt2 >= v2, col, _T), axis=-1, keepdims=True)
    m2 = col == i2

    e2 = jnp.exp(v2 - v1)
    p1 = 1.0 / (1.0 + e2)
    type_w = jnp.where(m1, p1, 0.0) + jnp.where(m2, 1.0 - p1, 0.0)

    es = jnp.exp(ls - jnp.max(ls, axis=-1, keepdims=True))
    soft = es / jnp.sum(es, axis=-1, keepdims=True)

    a = jax.nn.sigmoid(al[0, 0])
    wts = a * type_w + (1.0 - a) * soft

    out[...] = (x0[...] * wts[:, 0:1] + x1[...] * wts[:, 1:2]
                + x2[...] * wts[:, 2:3] + x3[...] * wts[:, 3:4])


def _tc_call(xs, w, bias, al):
    grid = (_NTC // _TB,)
    xspec = pl.BlockSpec((_TB, _D), lambda i: (i, 0))
    full = lambda *s: pl.BlockSpec(s, lambda i: tuple(0 for _ in s))
    return pl.pallas_call(
        _tc_body,
        grid=grid,
        in_specs=[xspec, xspec, xspec, xspec,
                  full(_T, _D, 2 * _T), full(1, 2 * _T), full(1, 1)],
        out_specs=xspec,
        out_shape=jax.ShapeDtypeStruct((_NTC, _D), jnp.float32),
        compiler_params=pltpu.CompilerParams(
            dimension_semantics=("arbitrary",)),
    )(xs[0], xs[1], xs[2], xs[3], w, bias, al)


_GDN = lax.GatherDimensionNumbers(
    offset_dims=(), collapsed_slice_dims=(0,), start_index_map=(0,))


def _lane_perm(v, idx):
    return lax.gather(v, idx, _GDN, (1,),
                      mode=lax.GatherScatterMode.PROMISE_IN_BOUNDS)


def _round_bf16(v):
    y = lax.bitcast_convert_type(v, jnp.int32)
    r = (y + 0x7FFF + ((y >> 16) & 1)) & jnp.int32(-65536)
    return lax.bitcast_convert_type(r, jnp.float32)


def _sc_body(x0, x1, x2, x3, w, params, out, xb, ob, wb, pb,
             si0, si1, so0, so1):
    wid = lax.axis_index("c") * 16 + lax.axis_index("s")
    pltpu.sync_copy(w, wb)
    pltpu.sync_copy(params, pb)
    lanes = lax.iota(jnp.int32, 16)
    perms = {sh: lax.reshape(lanes ^ sh, (16, 1)) for sh in (8, 4, 2, 1)}
    pv = pb[...]
    av = jnp.full((16,), 1.0, jnp.float32) * pv[8]
    a = 1.0 / (1.0 + jnp.exp(-av))
    one_m_a = 1.0 - a
    f32 = jnp.float32
    xsrc = (x0, x1, x2, x3)
    tok0 = wid * _TPW

    def start_in(g, par, sem):
        base = (_NTC + tok0 + g * _G) * _D
        for t in range(_T):
            pltpu.async_copy(xsrc[t].at[pl.ds(base, _GB)],
                             xb.at[pl.ds((par * _T + t) * _GB, _GB)], sem)

    def drain_in(par, sem):
        for t in range(_T):
            pltpu.make_async_copy(
                xsrc[t].at[pl.ds(0, _GB)],
                xb.at[pl.ds((par * _T + t) * _GB, _GB)], sem).wait()

    start_in(0, 0, si0)
    start_in(1, 1, si1)

    def run_group(g, par, si, so):
        drain_in(par, si)

        L = [jnp.zeros((16,), f32) for _ in range(8)]
        for blk in range(_G // _NT):
            def jbody(jj, accs):
                accs = list(accs)
                for t in range(_T):
                    wo = t * _CH * 128
                    wA = [wb[pl.ds(wo + (2 * jj) * 128 + k * 16, 16)]
                          for k in range(8)]
                    wB = [wb[pl.ds(wo + (2 * jj + 1) * 128 + k * 16, 16)]
                          for k in range(8)]
                    for n in range(_NT):
                        xo = ((par * _T + t) * _G + blk * _NT + n) * _D
                        r0 = _round_bf16(xb[pl.ds(xo + (2 * jj) * 16, 16)])
                        r1 = _round_bf16(xb[pl.ds(xo + (2 * jj + 1) * 16, 16)])
                        for k in range(8):
                            accs[n * 8 + k] = (accs[n * 8 + k]
                                               + r0 * wA[k] + r1 * wB[k])
                return tuple(accs)

            accs = lax.fori_loop(
                0, _CH // 2, jbody,
                tuple(jnp.zeros((16,), f32) for _ in range(_NT * 8)))
            for n in range(_NT):
                tok = blk * _NT + n
                for k in range(8):
                    s = accs[n * 8 + k]
                    for sh in (8, 4, 2, 1):
                        s = s + _lane_perm(s, perms[sh])
                    L[k] = jnp.where(lanes == tok, s, L[k])

        lt = [L[k] + pv[k] for k in range(4)]
        ls = [L[4 + k] + pv[4 + k] for k in range(4)]
        v1 = jnp.maximum(jnp.maximum(lt[0], lt[1]), jnp.maximum(lt[2], lt[3]))
        i1 = jnp.where(lt[0] >= v1, 0,
                       jnp.where(lt[1] >= v1, 1,
                                 jnp.where(lt[2] >= v1, 2, 3)))
        neg = jnp.float32(-3.0e38)
        l2 = [jnp.where(i1 == k, neg, lt[k]) for k in range(4)]
        v2 = jnp.maximum(jnp.maximum(l2[0], l2[1]), jnp.maximum(l2[2], l2[3]))
        i2 = jnp.where(l2[0] >= v2, 0,
                       jnp.where(l2[1] >= v2, 1,
                                 jnp.where(l2[2] >= v2, 2, 3)))
        e2 = jnp.exp(v2 - v1)
        p1 = 1.0 / (1.0 + e2)
        p2 = 1.0 - p1
        ms = jnp.maximum(jnp.maximum(ls[0], ls[1]), jnp.maximum(ls[2], ls[3]))
        es = [jnp.exp(ls[k] - ms) for k in range(4)]
        den = es[0] + es[1] + es[2] + es[3]
        wts = []
        for k in range(4):
            tw = (jnp.where(i1 == k, p1, 0.0) + jnp.where(i2 == k, p2, 0.0))
            wts.append(a * tw + one_m_a * (es[k] / den))

        obase = par * _GB
        for n in range(_G):
            s0 = wts[0][n]
            s1 = wts[1][n]
            s2 = wts[2][n]
            s3 = wts[3][n]
            xo0 = ((par * _T + 0) * _G + n) * _D
            xo1 = ((par * _T + 1) * _G + n) * _D
            xo2 = ((par * _T + 2) * _G + n) * _D
            xo3 = ((par * _T + 3) * _G + n) * _D

            def cbody(cc, carry2):
                for u in range(4):
                    o = (cc * 4 + u) * 16
                    ob[pl.ds(obase + n * _D + o, 16)] = (
                        xb[pl.ds(xo0 + o, 16)] * s0
                        + xb[pl.ds(xo1 + o, 16)] * s1
                        + xb[pl.ds(xo2 + o, 16)] * s2
                        + xb[pl.ds(xo3 + o, 16)] * s3)
                return carry2

            lax.fori_loop(0, _CH // 4, cbody, 0)

        base = (tok0 + g * _G) * _D
        pltpu.async_copy(ob.at[pl.ds(obase, _GB)],
                         out.at[pl.ds(base, _GB)], so)

    def pair(i, carry):
        for par, si, so in ((0, si0, so0), (1, si1, so1)):
            g = 2 * i + par

            @pl.when(i >= 1)
            def _():
                pltpu.make_async_copy(ob.at[pl.ds(par * _GB, _GB)],
                                      out.at[pl.ds(0, _GB)], so).wait()

            run_group(g, par, si, so)

            @pl.when(i < (_NG // 2) - 1)
            def _():
                start_in(g + 2, par, si)
        return carry

    lax.fori_loop(0, _NG // 2, pair, 0)

    for par, so in ((0, so0), (1, so1)):
        pltpu.make_async_copy(ob.at[pl.ds(par * _GB, _GB)],
                              out.at[pl.ds(0, _GB)], so).wait()


def _build_sc_call():
    mesh = plsc.VectorSubcoreMesh(core_axis_name="c", subcore_axis_name="s")
    return pl.kernel(
        _sc_body,
        mesh=mesh,
        out_type=jax.ShapeDtypeStruct((_NSC * _D,), jnp.float32),
        scratch_types=[
            pltpu.VMEM((2 * _T * _GB,), jnp.float32),
            pltpu.VMEM((2 * _GB,), jnp.float32),
            pltpu.VMEM((_T * _CH * 8 * 16,), jnp.float32),
            pltpu.VMEM((16,), jnp.float32),
            pltpu.SemaphoreType.DMA,
            pltpu.SemaphoreType.DMA,
            pltpu.SemaphoreType.DMA,
            pltpu.SemaphoreType.DMA,
        ],
    )


def kernel(mod0, mod1, mod2, mod3, W_top, b_top, W_soft, b_soft, alpha):
    B, S, D = mod0.shape
    N = B * S
    x2d = [m.reshape(N, D) for m in (mod0, mod1, mod2, mod3)]
    xfl = [m.reshape(N * D) for m in (mod0, mod1, mod2, mod3)]

    wt = W_top.reshape(_T, D, _T).transpose(2, 1, 0)
    ws = W_soft.reshape(_T, D, _T).transpose(2, 1, 0)
    w = jnp.concatenate([wt, ws], axis=-1)
    bias = jnp.concatenate([b_top, b_soft]).reshape(1, 2 * _T)
    al = alpha.reshape(1, 1)

    wsc = w.astype(jnp.bfloat16).astype(jnp.float32)
    wsc = wsc.reshape(_T, _CH, 16, 8).transpose(0, 1, 3, 2).reshape(-1)
    params = jnp.concatenate(
        [b_top, b_soft, alpha, jnp.zeros((7,), jnp.float32)])

    out_sc = _build_sc_call()(xfl[0], xfl[1], xfl[2], xfl[3], wsc, params)
    out_tc = _tc_call(x2d, w, bias, al)
    out = jnp.concatenate([out_tc.reshape(-1), out_sc])
    return out.reshape(B, S, D)


# speedup vs baseline: 4.2079x; 1.2003x over previous
import functools

import numpy as np
import jax
import jax.numpy as jnp
from jax import lax
from jax.experimental import pallas as pl
from jax.experimental.pallas import tpu as pltpu
from jax.experimental.pallas import tpu_sc as plsc

_T = 4
_D = 1024
_N = 8192
_NSC = 512
_NTC = _N - _NSC
_TB = 512

_NW = 32
_G = 8
_NT = 2
_CH = _D // 16
_TPW = _NSC // _NW
_NG = _TPW // _G
_GB = _G * _D


def _tc_body(x0, x1, x2, x3, w, bias, al, out):
    f32 = jnp.float32
    logits = (
        jax.lax.dot_general(x0[...], w[0], (((1,), (0,)), ((), ())),
                            preferred_element_type=f32)
        + jax.lax.dot_general(x1[...], w[1], (((1,), (0,)), ((), ())),
                              preferred_element_type=f32)
        + jax.lax.dot_general(x2[...], w[2], (((1,), (0,)), ((), ())),
                              preferred_element_type=f32)
        + jax.lax.dot_general(x3[...], w[3], (((1,), (0,)), ((), ())),
                              preferred_element_type=f32)
    ) + bias[0, :]
    lt = logits[:, :_T]
    ls = logits[:, _T:]

    col = jax.lax.broadcasted_iota(jnp.int32, (_TB, _T), 1)
    v1 = jnp.max(lt, axis=-1, keepdims=True)
    i1 = jnp.min(jnp.where(lt >= v1, col, _T), axis=-1, keepdims=True)
    m1 = col == i1
    lt2 = jnp.where(m1, -jnp.inf, lt)
    v2 = jnp.max(lt2, axis=-1, keepdims=True)
    i2 = jnp.min(jnp.where(lt2 >= v2, col, _T), axis=-1, keepdims=True)
    m2 = col == i2

    e2 = jnp.exp(v2 - v1)
    p1 = 1.0 / (1.0 + e2)
    type_w = jnp.where(m1, p1, 0.0) + jnp.where(m2, 1.0 - p1, 0.0)

    es = jnp.exp(ls - jnp.max(ls, axis=-1, keepdims=True))
    soft = es / jnp.sum(es, axis=-1, keepdims=True)

    a = jax.nn.sigmoid(al[0, 0])
    wts = a * type_w + (1.0 - a) * soft

    out[...] = (x0[...] * wts[:, 0:1] + x1[...] * wts[:, 1:2]
                + x2[...] * wts[:, 2:3] + x3[...] * wts[:, 3:4])


def _tc_call(xs, w, bias, al):
    grid = (_NTC // _TB,)
    xspec = pl.BlockSpec((_TB, _D), lambda i: (i, 0))
    full = lambda *s: pl.BlockSpec(s, lambda i: tuple(0 for _ in s))
    return pl.pallas_call(
        _tc_body,
        grid=grid,
        in_specs=[xspec, xspec, xspec, xspec,
                  full(_T, _D, 2 * _T), full(1, 2 * _T), full(1, 1)],
        out_specs=xspec,
        out_shape=jax.ShapeDtypeStruct((_NTC, _D), jnp.float32),
        compiler_params=pltpu.CompilerParams(
            dimension_semantics=("arbitrary",)),
    )(xs[0], xs[1], xs[2], xs[3], w, bias, al)


_GDN = lax.GatherDimensionNumbers(
    offset_dims=(), collapsed_slice_dims=(0,), start_index_map=(0,))


def _lane_perm(v, idx):
    return lax.gather(v, idx, _GDN, (1,),
                      mode=lax.GatherScatterMode.PROMISE_IN_BOUNDS)


def _round_bf16(v):
    y = lax.bitcast_convert_type(v, jnp.int32)
    r = (y + 0x7FFF + ((y >> 16) & 1)) & jnp.int32(-65536)
    return lax.bitcast_convert_type(r, jnp.float32)


def _sc_body(x0, x1, x2, x3, w, params, out, xb, ob, wb, pb,
             si0, si1, so0, so1):
    wid = lax.axis_index("c") * 16 + lax.axis_index("s")
    pltpu.sync_copy(w, wb)
    pltpu.sync_copy(params, pb)
    lanes = lax.iota(jnp.int32, 16)
    perms = {sh: lax.reshape(lanes ^ sh, (16, 1)) for sh in (8, 4, 2, 1)}
    pv = pb[...]
    av = jnp.full((16,), 1.0, jnp.float32) * pv[8]
    a = 1.0 / (1.0 + jnp.exp(-av))
    one_m_a = 1.0 - a
    f32 = jnp.float32
    xsrc = (x0, x1, x2, x3)
    tok0 = wid * _TPW

    def start_in(g, par, sem):
        base = (tok0 + g * _G) * _D
        for t in range(_T):
            pltpu.async_copy(xsrc[t].at[pl.ds(base, _GB)],
                             xb.at[pl.ds((par * _T + t) * _GB, _GB)], sem)

    def drain_in(par, sem):
        for t in range(_T):
            pltpu.make_async_copy(
                xsrc[t].at[pl.ds(0, _GB)],
                xb.at[pl.ds((par * _T + t) * _GB, _GB)], sem).wait()

    start_in(0, 0, si0)
    start_in(1, 1, si1)

    def run_group(g, par, si, so):
        drain_in(par, si)

        L = [jnp.zeros((16,), f32) for _ in range(8)]
        for blk in range(_G // _NT):
            def jbody(jj, accs):
                accs = list(accs)
                for t in range(_T):
                    wo = t * _CH * 128
                    wA = [wb[pl.ds(wo + (2 * jj) * 128 + k * 16, 16)]
                          for k in range(8)]
                    wB = [wb[pl.ds(wo + (2 * jj + 1) * 128 + k * 16, 16)]
                          for k in range(8)]
                    for n in range(_NT):
                        xo = ((par * _T + t) * _G + blk * _NT + n) * _D
                        r0 = _round_bf16(xb[pl.ds(xo + (2 * jj) * 16, 16)])
                        r1 = _round_bf16(xb[pl.ds(xo + (2 * jj + 1) * 16, 16)])
                        for k in range(8):
                            accs[n * 8 + k] = (accs[n * 8 + k]
                                               + r0 * wA[k] + r1 * wB[k])
                return tuple(accs)

            accs = lax.fori_loop(
                0, _CH // 2, jbody,
                tuple(jnp.zeros((16,), f32) for _ in range(_NT * 8)))
            for n in range(_NT):
                tok = blk * _NT + n
                for k in range(8):
                    s = accs[n * 8 + k]
                    for sh in (8, 4, 2, 1):
                        s = s + _lane_perm(s, perms[sh])
                    L[k] = jnp.where(lanes == tok, s, L[k])

        lt = [L[k] + pv[k] for k in range(4)]
        ls = [L[4 + k] + pv[4 + k] for k in range(4)]
        v1 = jnp.maximum(jnp.maximum(lt[0], lt[1]), jnp.maximum(lt[2], lt[3]))
        i1 = jnp.where(lt[0] >= v1, 0,
                       jnp.where(lt[1] >= v1, 1,
                                 jnp.where(lt[2] >= v1, 2, 3)))
        neg = jnp.float32(-3.0e38)
        l2 = [jnp.where(i1 == k, neg, lt[k]) for k in range(4)]
        v2 = jnp.maximum(jnp.maximum(l2[0], l2[1]), jnp.maximum(l2[2], l2[3]))
        i2 = jnp.where(l2[0] >= v2, 0,
                       jnp.where(l2[1] >= v2, 1,
                                 jnp.where(l2[2] >= v2, 2, 3)))
        e2 = jnp.exp(v2 - v1)
        p1 = 1.0 / (1.0 + e2)
        p2 = 1.0 - p1
        ms = jnp.maximum(jnp.maximum(ls[0], ls[1]), jnp.maximum(ls[2], ls[3]))
        es = [jnp.exp(ls[k] - ms) for k in range(4)]
        den = es[0] + es[1] + es[2] + es[3]
        wts = []
        for k in range(4):
            tw = (jnp.where(i1 == k, p1, 0.0) + jnp.where(i2 == k, p2, 0.0))
            wts.append(a * tw + one_m_a * (es[k] / den))

        obase = par * _GB
        for n in range(_G):
            s0 = wts[0][n]
            s1 = wts[1][n]
            s2 = wts[2][n]
            s3 = wts[3][n]
            xo0 = ((par * _T + 0) * _G + n) * _D
            xo1 = ((par * _T + 1) * _G + n) * _D
            xo2 = ((par * _T + 2) * _G + n) * _D
            xo3 = ((par * _T + 3) * _G + n) * _D

            def cbody(cc, carry2):
                for u in range(4):
                    o = (cc * 4 + u) * 16
                    ob[pl.ds(obase + n * _D + o, 16)] = (
                        xb[pl.ds(xo0 + o, 16)] * s0
                        + xb[pl.ds(xo1 + o, 16)] * s1
                        + xb[pl.ds(xo2 + o, 16)] * s2
                        + xb[pl.ds(xo3 + o, 16)] * s3)
                return carry2

            lax.fori_loop(0, _CH // 4, cbody, 0)

        base = (tok0 + g * _G) * _D
        pltpu.async_copy(ob.at[pl.ds(obase, _GB)],
                         out.at[pl.ds(base, _GB)], so)

    def pair(i, carry):
        for par, si, so in ((0, si0, so0), (1, si1, so1)):
            g = 2 * i + par

            @pl.when(i >= 1)
            def _():
                pltpu.make_async_copy(ob.at[pl.ds(par * _GB, _GB)],
                                      out.at[pl.ds(0, _GB)], so).wait()

            run_group(g, par, si, so)

            @pl.when(i < (_NG // 2) - 1)
            def _():
                start_in(g + 2, par, si)
        return carry

    lax.fori_loop(0, _NG // 2, pair, 0)

    for par, so in ((0, so0), (1, so1)):
        pltpu.make_async_copy(ob.at[pl.ds(par * _GB, _GB)],
                              out.at[pl.ds(0, _GB)], so).wait()


def _build_sc_call():
    mesh = plsc.VectorSubcoreMesh(core_axis_name="c", subcore_axis_name="s")
    return pl.kernel(
        _sc_body,
        mesh=mesh,
        out_type=jax.ShapeDtypeStruct((_NSC * _D,), jnp.float32),
        scratch_types=[
            pltpu.VMEM((2 * _T * _GB,), jnp.float32),
            pltpu.VMEM((2 * _GB,), jnp.float32),
            pltpu.VMEM((_T * _CH * 8 * 16,), jnp.float32),
            pltpu.VMEM((16,), jnp.float32),
            pltpu.SemaphoreType.DMA,
            pltpu.SemaphoreType.DMA,
            pltpu.SemaphoreType.DMA,
            pltpu.SemaphoreType.DMA,
        ],
    )


def kernel(mod0, mod1, mod2, mod3, W_top, b_top, W_soft, b_soft, alpha):
    B, S, D = mod0.shape
    N = B * S
    x2d = [m.reshape(N, D) for m in (mod0, mod1, mod2, mod3)]
    xfl = [m.reshape(N, D)[_NTC:].reshape(_NSC * D)
           for m in (mod0, mod1, mod2, mod3)]

    wt = W_top.reshape(_T, D, _T).transpose(2, 1, 0)
    ws = W_soft.reshape(_T, D, _T).transpose(2, 1, 0)
    w = jnp.concatenate([wt, ws], axis=-1)
    bias = jnp.concatenate([b_top, b_soft]).reshape(1, 2 * _T)
    al = alpha.reshape(1, 1)

    wsc = w.astype(jnp.bfloat16).astype(jnp.float32)
    wsc = wsc.reshape(_T, _CH, 16, 8).transpose(0, 1, 3, 2).reshape(-1)
    params = jnp.concatenate(
        [b_top, b_soft, alpha, jnp.zeros((7,), jnp.float32)])

    out_sc = _build_sc_call()(xfl[0], xfl[1], xfl[2], xfl[3], wsc, params)
    out_tc = _tc_call(x2d, w, bias, al)
    out = jnp.concatenate([out_tc.reshape(-1), out_sc])
    return out.reshape(B, S, D)


# speedup vs baseline: 4.4402x; 1.0552x over previous
import functools

import numpy as np
import jax
import jax.numpy as jnp
from jax import lax
from jax.experimental import pallas as pl
from jax.experimental.pallas import tpu as pltpu
from jax.experimental.pallas import tpu_sc as plsc

_T = 4
_D = 1024
_N = 8192
_NSC = 256
_NTC = _N - _NSC
_TB = 256

_NW = 32
_G = 4
_NT = 2
_CH = _D // 16
_TPW = _NSC // _NW
_NG = _TPW // _G
_GB = _G * _D


def _tc_body(x0, x1, x2, x3, w, bias, al, out):
    f32 = jnp.float32
    logits = (
        jax.lax.dot_general(x0[...], w[0], (((1,), (0,)), ((), ())),
                            preferred_element_type=f32)
        + jax.lax.dot_general(x1[...], w[1], (((1,), (0,)), ((), ())),
                              preferred_element_type=f32)
        + jax.lax.dot_general(x2[...], w[2], (((1,), (0,)), ((), ())),
                              preferred_element_type=f32)
        + jax.lax.dot_general(x3[...], w[3], (((1,), (0,)), ((), ())),
                              preferred_element_type=f32)
    ) + bias[0, :]
    lt = logits[:, :_T]
    ls = logits[:, _T:]

    col = jax.lax.broadcasted_iota(jnp.int32, (_TB, _T), 1)
    v1 = jnp.max(lt, axis=-1, keepdims=True)
    i1 = jnp.min(jnp.where(lt >= v1, col, _T), axis=-1, keepdims=True)
    m1 = col == i1
    lt2 = jnp.where(m1, -jnp.inf, lt)
    v2 = jnp.max(lt2, axis=-1, keepdims=True)
    i2 = jnp.min(jnp.where(lt2 >= v2, col, _T), axis=-1, keepdims=True)
    m2 = col == i2

    e2 = jnp.exp(v2 - v1)
    p1 = 1.0 / (1.0 + e2)
    type_w = jnp.where(m1, p1, 0.0) + jnp.where(m2, 1.0 - p1, 0.0)

    es = jnp.exp(ls - jnp.max(ls, axis=-1, keepdims=True))
    soft = es / jnp.sum(es, axis=-1, keepdims=True)

    a = jax.nn.sigmoid(al[0, 0])
    wts = a * type_w + (1.0 - a) * soft

    out[...] = (x0[...] * wts[:, 0:1] + x1[...] * wts[:, 1:2]
                + x2[...] * wts[:, 2:3] + x3[...] * wts[:, 3:4])


def _tc_call(xs, w, bias, al):
    grid = (_NTC // _TB,)
    xspec = pl.BlockSpec((_TB, _D), lambda i: (i, 0))
    full = lambda *s: pl.BlockSpec(s, lambda i: tuple(0 for _ in s))
    return pl.pallas_call(
        _tc_body,
        grid=grid,
        in_specs=[xspec, xspec, xspec, xspec,
                  full(_T, _D, 2 * _T), full(1, 2 * _T), full(1, 1)],
        out_specs=xspec,
        out_shape=jax.ShapeDtypeStruct((_NTC, _D), jnp.float32),
        compiler_params=pltpu.CompilerParams(
            dimension_semantics=("arbitrary",)),
    )(xs[0], xs[1], xs[2], xs[3], w, bias, al)


_GDN = lax.GatherDimensionNumbers(
    offset_dims=(), collapsed_slice_dims=(0,), start_index_map=(0,))


def _lane_perm(v, idx):
    return lax.gather(v, idx, _GDN, (1,),
                      mode=lax.GatherScatterMode.PROMISE_IN_BOUNDS)


def _round_bf16(v):
    y = lax.bitcast_convert_type(v, jnp.int32)
    r = (y + 0x7FFF + ((y >> 16) & 1)) & jnp.int32(-65536)
    return lax.bitcast_convert_type(r, jnp.float32)


def _sc_body(x0, x1, x2, x3, w, params, out, xb, ob, wb, pb,
             si0, si1, so0, so1):
    wid = lax.axis_index("c") * 16 + lax.axis_index("s")
    pltpu.sync_copy(w, wb)
    pltpu.sync_copy(params, pb)
    lanes = lax.iota(jnp.int32, 16)
    perms = {sh: lax.reshape(lanes ^ sh, (16, 1)) for sh in (8, 4, 2, 1)}
    pv = pb[...]
    av = jnp.full((16,), 1.0, jnp.float32) * pv[8]
    a = 1.0 / (1.0 + jnp.exp(-av))
    one_m_a = 1.0 - a
    f32 = jnp.float32
    xsrc = (x0, x1, x2, x3)
    tok0 = wid * _TPW

    def start_in(g, par, sem):
        base = (tok0 + g * _G) * _D
        for t in range(_T):
            pltpu.async_copy(xsrc[t].at[pl.ds(base, _GB)],
                             xb.at[pl.ds((par * _T + t) * _GB, _GB)], sem)

    def drain_in(par, sem):
        for t in range(_T):
            pltpu.make_async_copy(
                xsrc[t].at[pl.ds(0, _GB)],
                xb.at[pl.ds((par * _T + t) * _GB, _GB)], sem).wait()

    start_in(0, 0, si0)
    start_in(1, 1, si1)

    def run_group(g, par, si, so):
        drain_in(par, si)

        L = [jnp.zeros((16,), f32) for _ in range(8)]
        for blk in range(_G // _NT):
            def jbody(jj, accs):
                accs = list(accs)
                for t in range(_T):
                    wo = t * _CH * 128
                    wA = [wb[pl.ds(wo + (2 * jj) * 128 + k * 16, 16)]
                          for k in range(8)]
                    wB = [wb[pl.ds(wo + (2 * jj + 1) * 128 + k * 16, 16)]
                          for k in range(8)]
                    for n in range(_NT):
                        xo = ((par * _T + t) * _G + blk * _NT + n) * _D
                        r0 = _round_bf16(xb[pl.ds(xo + (2 * jj) * 16, 16)])
                        r1 = _round_bf16(xb[pl.ds(xo + (2 * jj + 1) * 16, 16)])
                        for k in range(8):
                            accs[n * 8 + k] = (accs[n * 8 + k]
                                               + r0 * wA[k] + r1 * wB[k])
                return tuple(accs)

            accs = lax.fori_loop(
                0, _CH // 2, jbody,
                tuple(jnp.zeros((16,), f32) for _ in range(_NT * 8)))
            for n in range(_NT):
                tok = blk * _NT + n
                for k in range(8):
                    s = accs[n * 8 + k]
                    for sh in (8, 4, 2, 1):
                        s = s + _lane_perm(s, perms[sh])
                    L[k] = jnp.where(lanes == tok, s, L[k])

        lt = [L[k] + pv[k] for k in range(4)]
        ls = [L[4 + k] + pv[4 + k] for k in range(4)]
        v1 = jnp.maximum(jnp.maximum(lt[0], lt[1]), jnp.maximum(lt[2], lt[3]))
        i1 = jnp.where(lt[0] >= v1, 0,
                       jnp.where(lt[1] >= v1, 1,
                                 jnp.where(lt[2] >= v1, 2, 3)))
        neg = jnp.float32(-3.0e38)
        l2 = [jnp.where(i1 == k, neg, lt[k]) for k in range(4)]
        v2 = jnp.maximum(jnp.maximum(l2[0], l2[1]), jnp.maximum(l2[2], l2[3]))
        i2 = jnp.where(l2[0] >= v2, 0,
                       jnp.where(l2[1] >= v2, 1,
                                 jnp.where(l2[2] >= v2, 2, 3)))
        e2 = jnp.exp(v2 - v1)
        p1 = 1.0 / (1.0 + e2)
        p2 = 1.0 - p1
        ms = jnp.maximum(jnp.maximum(ls[0], ls[1]), jnp.maximum(ls[2], ls[3]))
        es = [jnp.exp(ls[k] - ms) for k in range(4)]
        den = es[0] + es[1] + es[2] + es[3]
        wts = []
        for k in range(4):
            tw = (jnp.where(i1 == k, p1, 0.0) + jnp.where(i2 == k, p2, 0.0))
            wts.append(a * tw + one_m_a * (es[k] / den))

        obase = par * _GB
        for n in range(_G):
            s0 = wts[0][n]
            s1 = wts[1][n]
            s2 = wts[2][n]
            s3 = wts[3][n]
            xo0 = ((par * _T + 0) * _G + n) * _D
            xo1 = ((par * _T + 1) * _G + n) * _D
            xo2 = ((par * _T + 2) * _G + n) * _D
            xo3 = ((par * _T + 3) * _G + n) * _D

            def cbody(cc, carry2):
                for u in range(4):
                    o = (cc * 4 + u) * 16
                    ob[pl.ds(obase + n * _D + o, 16)] = (
                        xb[pl.ds(xo0 + o, 16)] * s0
                        + xb[pl.ds(xo1 + o, 16)] * s1
                        + xb[pl.ds(xo2 + o, 16)] * s2
                        + xb[pl.ds(xo3 + o, 16)] * s3)
                return carry2

            lax.fori_loop(0, _CH // 4, cbody, 0)

        base = (tok0 + g * _G) * _D
        pltpu.async_copy(ob.at[pl.ds(obase, _GB)],
                         out.at[pl.ds(base, _GB)], so)

    def pair(i, carry):
        for par, si, so in ((0, si0, so0), (1, si1, so1)):
            g = 2 * i + par

            @pl.when(i >= 1)
            def _():
                pltpu.make_async_copy(ob.at[pl.ds(par * _GB, _GB)],
                                      out.at[pl.ds(0, _GB)], so).wait()

            run_group(g, par, si, so)

            @pl.when(i < (_NG // 2) - 1)
            def _():
                start_in(g + 2, par, si)
        return carry

    lax.fori_loop(0, _NG // 2, pair, 0)

    for par, so in ((0, so0), (1, so1)):
        pltpu.make_async_copy(ob.at[pl.ds(par * _GB, _GB)],
                              out.at[pl.ds(0, _GB)], so).wait()


def _build_sc_call():
    mesh = plsc.VectorSubcoreMesh(core_axis_name="c", subcore_axis_name="s")
    return pl.kernel(
        _sc_body,
        mesh=mesh,
        out_type=jax.ShapeDtypeStruct((_NSC * _D,), jnp.float32),
        scratch_types=[
            pltpu.VMEM((2 * _T * _GB,), jnp.float32),
            pltpu.VMEM((2 * _GB,), jnp.float32),
            pltpu.VMEM((_T * _CH * 8 * 16,), jnp.float32),
            pltpu.VMEM((16,), jnp.float32),
            pltpu.SemaphoreType.DMA,
            pltpu.SemaphoreType.DMA,
            pltpu.SemaphoreType.DMA,
            pltpu.SemaphoreType.DMA,
        ],
    )


def kernel(mod0, mod1, mod2, mod3, W_top, b_top, W_soft, b_soft, alpha):
    B, S, D = mod0.shape
    N = B * S
    x2d = [m.reshape(N, D) for m in (mod0, mod1, mod2, mod3)]
    xfl = [m.reshape(N, D)[_NTC:].reshape(_NSC * D)
           for m in (mod0, mod1, mod2, mod3)]

    wt = W_top.reshape(_T, D, _T).transpose(2, 1, 0)
    ws = W_soft.reshape(_T, D, _T).transpose(2, 1, 0)
    w = jnp.concatenate([wt, ws], axis=-1)
    bias = jnp.concatenate([b_top, b_soft]).reshape(1, 2 * _T)
    al = alpha.reshape(1, 1)

    wsc = w.astype(jnp.bfloat16).astype(jnp.float32)
    wsc = wsc.reshape(_T, _CH, 16, 8).transpose(0, 1, 3, 2).reshape(-1)
    params = jnp.concatenate(
        [b_top, b_soft, alpha, jnp.zeros((7,), jnp.float32)])

    out_sc = _build_sc_call()(xfl[0], xfl[1], xfl[2], xfl[3], wsc, params)
    out_tc = _tc_call(x2d, w, bias, al)
    out = jnp.concatenate([out_tc.reshape(-1), out_sc])
    return out.reshape(B, S, D)


# speedup vs baseline: 4.4894x; 1.0111x over previous
import functools

import numpy as np
import jax
import jax.numpy as jnp
from jax import lax
from jax.experimental import pallas as pl
from jax.experimental.pallas import tpu as pltpu
from jax.experimental.pallas import tpu_sc as plsc

_T = 4
_D = 1024
_N = 8192
_NSC = 256
_NTC = _N - _NSC
_TB = 256

_NW = 32
_G = 4
_NT = 2
_CH = _D // 16
_TPW = _NSC // _NW
_NG = _TPW // _G
_GB = _G * _D


def _tc_body(x0, x1, x2, x3, w, bias, al, out):
    f32 = jnp.float32
    logits = (
        jax.lax.dot_general(x0[...], w[0], (((1,), (0,)), ((), ())),
                            preferred_element_type=f32)
        + jax.lax.dot_general(x1[...], w[1], (((1,), (0,)), ((), ())),
                              preferred_element_type=f32)
        + jax.lax.dot_general(x2[...], w[2], (((1,), (0,)), ((), ())),
                              preferred_element_type=f32)
        + jax.lax.dot_general(x3[...], w[3], (((1,), (0,)), ((), ())),
                              preferred_element_type=f32)
    ) + bias[0, :]
    lt = logits[:, :_T]
    ls = logits[:, _T:]

    col = jax.lax.broadcasted_iota(jnp.int32, (_TB, _T), 1)
    v1 = jnp.max(lt, axis=-1, keepdims=True)
    i1 = jnp.min(jnp.where(lt >= v1, col, _T), axis=-1, keepdims=True)
    m1 = col == i1
    lt2 = jnp.where(m1, -jnp.inf, lt)
    v2 = jnp.max(lt2, axis=-1, keepdims=True)
    i2 = jnp.min(jnp.where(lt2 >= v2, col, _T), axis=-1, keepdims=True)
    m2 = col == i2

    e2 = jnp.exp(v2 - v1)
    p1 = 1.0 / (1.0 + e2)
    type_w = jnp.where(m1, p1, 0.0) + jnp.where(m2, 1.0 - p1, 0.0)

    es = jnp.exp(ls - jnp.max(ls, axis=-1, keepdims=True))
    soft = es / jnp.sum(es, axis=-1, keepdims=True)

    a = jax.nn.sigmoid(al[0, 0])
    wts = a * type_w + (1.0 - a) * soft

    out[...] = (x0[...] * wts[:, 0:1] + x1[...] * wts[:, 1:2]
                + x2[...] * wts[:, 2:3] + x3[...] * wts[:, 3:4])


def _tc_call(xs, w, bias, al):
    grid = (_NTC // _TB,)
    xspec = pl.BlockSpec((_TB, _D), lambda i: (i, 0))
    full = lambda *s: pl.BlockSpec(s, lambda i: tuple(0 for _ in s))
    return pl.pallas_call(
        _tc_body,
        grid=grid,
        in_specs=[xspec, xspec, xspec, xspec,
                  full(_T, _D, 2 * _T), full(1, 2 * _T), full(1, 1)],
        out_specs=xspec,
        out_shape=jax.ShapeDtypeStruct((_NTC, _D), jnp.float32),
        compiler_params=pltpu.CompilerParams(
            dimension_semantics=("arbitrary",)),
    )(xs[0], xs[1], xs[2], xs[3], w, bias, al)


_GDN = lax.GatherDimensionNumbers(
    offset_dims=(), collapsed_slice_dims=(0,), start_index_map=(0,))


def _lane_perm(v, idx):
    return lax.gather(v, idx, _GDN, (1,),
                      mode=lax.GatherScatterMode.PROMISE_IN_BOUNDS)


def _round_bf16(v):
    y = lax.bitcast_convert_type(v, jnp.int32)
    r = (y + 0x7FFF + ((y >> 16) & 1)) & jnp.int32(-65536)
    return lax.bitcast_convert_type(r, jnp.float32)


def _sc_body(x, w, out, xb, ob, wb,
             si0, si1, so0, so1):
    wid = lax.axis_index("c") * 16 + lax.axis_index("s")
    pltpu.sync_copy(w, wb)
    lanes = lax.iota(jnp.int32, 16)
    perms = {sh: lax.reshape(lanes ^ sh, (16, 1)) for sh in (8, 4, 2, 1)}
    pv = wb[pl.ds(_T * _CH * 128, 16)]
    av = jnp.full((16,), 1.0, jnp.float32) * pv[8]
    a = 1.0 / (1.0 + jnp.exp(-av))
    one_m_a = 1.0 - a
    f32 = jnp.float32
    tok0 = wid * _TPW

    def start_in(g, par, sem):
        base = (tok0 + g * _G) * _D
        for t in range(_T):
            pltpu.async_copy(
                x.at[pl.ds(t * _NSC * _D + base, _GB)],
                xb.at[pl.ds((par * _T + t) * _GB, _GB)], sem)

    def drain_in(par, sem):
        for t in range(_T):
            pltpu.make_async_copy(
                x.at[pl.ds(0, _GB)],
                xb.at[pl.ds((par * _T + t) * _GB, _GB)], sem).wait()

    start_in(0, 0, si0)
    start_in(1, 1, si1)

    def run_group(g, par, si, so):
        drain_in(par, si)

        L = [jnp.zeros((16,), f32) for _ in range(8)]
        for blk in range(_G // _NT):
            def jbody(jj, accs):
                accs = list(accs)
                for t in range(_T):
                    wo = t * _CH * 128
                    wA = [wb[pl.ds(wo + (2 * jj) * 128 + k * 16, 16)]
                          for k in range(8)]
                    wB = [wb[pl.ds(wo + (2 * jj + 1) * 128 + k * 16, 16)]
                          for k in range(8)]
                    for n in range(_NT):
                        xo = ((par * _T + t) * _G + blk * _NT + n) * _D
                        r0 = _round_bf16(xb[pl.ds(xo + (2 * jj) * 16, 16)])
                        r1 = _round_bf16(xb[pl.ds(xo + (2 * jj + 1) * 16, 16)])
                        for k in range(8):
                            accs[n * 8 + k] = (accs[n * 8 + k]
                                               + r0 * wA[k] + r1 * wB[k])
                return tuple(accs)

            accs = lax.fori_loop(
                0, _CH // 2, jbody,
                tuple(jnp.zeros((16,), f32) for _ in range(_NT * 8)))
            for n in range(_NT):
                tok = blk * _NT + n
                for k in range(8):
                    s = accs[n * 8 + k]
                    for sh in (8, 4, 2, 1):
                        s = s + _lane_perm(s, perms[sh])
                    L[k] = jnp.where(lanes == tok, s, L[k])

        lt = [L[k] + pv[k] for k in range(4)]
        ls = [L[4 + k] + pv[4 + k] for k in range(4)]
        v1 = jnp.maximum(jnp.maximum(lt[0], lt[1]), jnp.maximum(lt[2], lt[3]))
        i1 = jnp.where(lt[0] >= v1, 0,
                       jnp.where(lt[1] >= v1, 1,
                                 jnp.where(lt[2] >= v1, 2, 3)))
        neg = jnp.float32(-3.0e38)
        l2 = [jnp.where(i1 == k, neg, lt[k]) for k in range(4)]
        v2 = jnp.maximum(jnp.maximum(l2[0], l2[1]), jnp.maximum(l2[2], l2[3]))
        i2 = jnp.where(l2[0] >= v2, 0,
                       jnp.where(l2[1] >= v2, 1,
                                 jnp.where(l2[2] >= v2, 2, 3)))
        e2 = jnp.exp(v2 - v1)
        p1 = 1.0 / (1.0 + e2)
        p2 = 1.0 - p1
        ms = jnp.maximum(jnp.maximum(ls[0], ls[1]), jnp.maximum(ls[2], ls[3]))
        es = [jnp.exp(ls[k] - ms) for k in range(4)]
        den = es[0] + es[1] + es[2] + es[3]
        wts = []
        for k in range(4):
            tw = (jnp.where(i1 == k, p1, 0.0) + jnp.where(i2 == k, p2, 0.0))
            wts.append(a * tw + one_m_a * (es[k] / den))

        obase = par * _GB
        for n in range(_G):
            s0 = wts[0][n]
            s1 = wts[1][n]
            s2 = wts[2][n]
            s3 = wts[3][n]
            xo0 = ((par * _T + 0) * _G + n) * _D
            xo1 = ((par * _T + 1) * _G + n) * _D
            xo2 = ((par * _T + 2) * _G + n) * _D
            xo3 = ((par * _T + 3) * _G + n) * _D

            def cbody(cc, carry2):
                for u in range(4):
                    o = (cc * 4 + u) * 16
                    ob[pl.ds(obase + n * _D + o, 16)] = (
                        xb[pl.ds(xo0 + o, 16)] * s0
                        + xb[pl.ds(xo1 + o, 16)] * s1
                        + xb[pl.ds(xo2 + o, 16)] * s2
                        + xb[pl.ds(xo3 + o, 16)] * s3)
                return carry2

            lax.fori_loop(0, _CH // 4, cbody, 0)

        base = (tok0 + g * _G) * _D
        pltpu.async_copy(ob.at[pl.ds(obase, _GB)],
                         out.at[pl.ds(base, _GB)], so)

    def pair(i, carry):
        for par, si, so in ((0, si0, so0), (1, si1, so1)):
            g = 2 * i + par

            @pl.when(i >= 1)
            def _():
                pltpu.make_async_copy(ob.at[pl.ds(par * _GB, _GB)],
                                      out.at[pl.ds(0, _GB)], so).wait()

            run_group(g, par, si, so)

            @pl.when(i < (_NG // 2) - 1)
            def _():
                start_in(g + 2, par, si)
        return carry

    lax.fori_loop(0, _NG // 2, pair, 0)

    for par, so in ((0, so0), (1, so1)):
        pltpu.make_async_copy(ob.at[pl.ds(par * _GB, _GB)],
                              out.at[pl.ds(0, _GB)], so).wait()


def _build_sc_call():
    mesh = plsc.VectorSubcoreMesh(core_axis_name="c", subcore_axis_name="s")
    return pl.kernel(
        _sc_body,
        mesh=mesh,
        out_type=jax.ShapeDtypeStruct((_NSC * _D,), jnp.float32),
        scratch_types=[
            pltpu.VMEM((2 * _T * _GB,), jnp.float32),
            pltpu.VMEM((2 * _GB,), jnp.float32),
            pltpu.VMEM((_T * _CH * 128 + 16,), jnp.float32),
            pltpu.SemaphoreType.DMA,
            pltpu.SemaphoreType.DMA,
            pltpu.SemaphoreType.DMA,
            pltpu.SemaphoreType.DMA,
        ],
    )


def kernel(mod0, mod1, mod2, mod3, W_top, b_top, W_soft, b_soft, alpha):
    B, S, D = mod0.shape
    N = B * S
    x2d = [m.reshape(N, D) for m in (mod0, mod1, mod2, mod3)]
    xfl = [m.reshape(N, D)[_NTC:].reshape(_NSC * D)
           for m in (mod0, mod1, mod2, mod3)]

    wt = W_top.reshape(_T, D, _T).transpose(2, 1, 0)
    ws = W_soft.reshape(_T, D, _T).transpose(2, 1, 0)
    w = jnp.concatenate([wt, ws], axis=-1)
    bias = jnp.concatenate([b_top, b_soft]).reshape(1, 2 * _T)
    al = alpha.reshape(1, 1)

    wsc = w.astype(jnp.bfloat16).astype(jnp.float32)
    wsc = wsc.reshape(_T, _CH, 16, 8).transpose(0, 1, 3, 2).reshape(-1)
    params = jnp.concatenate(
        [b_top, b_soft, alpha, jnp.zeros((7,), jnp.float32)])
    wp = jnp.concatenate([wsc, params])
    xcat = jnp.concatenate(xfl)

    out_sc = _build_sc_call()(xcat, wp)
    out_tc = _tc_call(x2d, w, bias, al)
    out = jnp.concatenate([out_tc.reshape(-1), out_sc])
    return out.reshape(B, S, D)


# speedup vs baseline: 4.4928x; 1.0008x over previous
import jax
import jax.numpy as jnp
from jax import lax
from jax.experimental import pallas as pl
from jax.experimental.pallas import tpu as pltpu
from jax.experimental.pallas import tpu_sc as plsc

_T = 4
_D = 1024
_N = 8192
_NSC = 256
_NTC = _N - _NSC
_TB = 256

_NW = 32
_G = 4
_NT = 2
_CH = _D // 16
_TPW = _NSC // _NW
_NG = _TPW // _G
_GB = _G * _D


def _tc_body(x0, x1, x2, x3, w, bias, al, out):
    f32 = jnp.float32
    logits = (
        jax.lax.dot_general(x0[...], w[0], (((1,), (0,)), ((), ())),
                            preferred_element_type=f32)
        + jax.lax.dot_general(x1[...], w[1], (((1,), (0,)), ((), ())),
                              preferred_element_type=f32)
        + jax.lax.dot_general(x2[...], w[2], (((1,), (0,)), ((), ())),
                              preferred_element_type=f32)
        + jax.lax.dot_general(x3[...], w[3], (((1,), (0,)), ((), ())),
                              preferred_element_type=f32)
    ) + bias[0, :]
    lt = logits[:, :_T]
    ls = logits[:, _T:]

    col = jax.lax.broadcasted_iota(jnp.int32, (_TB, _T), 1)
    v1 = jnp.max(lt, axis=-1, keepdims=True)
    i1 = jnp.min(jnp.where(lt >= v1, col, _T), axis=-1, keepdims=True)
    m1 = col == i1
    lt2 = jnp.where(m1, -jnp.inf, lt)
    v2 = jnp.max(lt2, axis=-1, keepdims=True)
    i2 = jnp.min(jnp.where(lt2 >= v2, col, _T), axis=-1, keepdims=True)
    m2 = col == i2

    e2 = jnp.exp(v2 - v1)
    p1 = 1.0 / (1.0 + e2)
    type_w = jnp.where(m1, p1, 0.0) + jnp.where(m2, 1.0 - p1, 0.0)

    es = jnp.exp(ls - jnp.max(ls, axis=-1, keepdims=True))
    soft = es / jnp.sum(es, axis=-1, keepdims=True)

    a = jax.nn.sigmoid(al[0, 0])
    wts = a * type_w + (1.0 - a) * soft

    out[...] = (x0[...] * wts[:, 0:1] + x1[...] * wts[:, 1:2]
                + x2[...] * wts[:, 2:3] + x3[...] * wts[:, 3:4])


def _tc_call(xs, w, bias, al):
    grid = (_NTC // _TB,)
    xspec = pl.BlockSpec((_TB, _D), lambda i: (i, 0))
    full = lambda *s: pl.BlockSpec(s, lambda i: tuple(0 for _ in s))
    return pl.pallas_call(
        _tc_body,
        grid=grid,
        in_specs=[xspec, xspec, xspec, xspec,
                  full(_T, _D, 2 * _T), full(1, 2 * _T), full(1, 1)],
        out_specs=xspec,
        out_shape=jax.ShapeDtypeStruct((_NTC, _D), jnp.float32),
        compiler_params=pltpu.CompilerParams(
            dimension_semantics=("arbitrary",)),
    )(xs[0], xs[1], xs[2], xs[3], w, bias, al)


_GDN = lax.GatherDimensionNumbers(
    offset_dims=(), collapsed_slice_dims=(0,), start_index_map=(0,))


def _lane_perm(v, idx):
    return lax.gather(v, idx, _GDN, (1,),
                      mode=lax.GatherScatterMode.PROMISE_IN_BOUNDS)


def _round_bf16(v):
    y = lax.bitcast_convert_type(v, jnp.int32)
    r = (y + 0x7FFF + ((y >> 16) & 1)) & jnp.int32(-65536)
    return lax.bitcast_convert_type(r, jnp.float32)


def _sc_body(x, w, out, xb, ob, wb,
             si0, si1, so0, so1):
    wid = lax.axis_index("c") * 16 + lax.axis_index("s")
    pltpu.sync_copy(w, wb)
    lanes = lax.iota(jnp.int32, 16)
    perms = {sh: lax.reshape(lanes ^ sh, (16, 1)) for sh in (8, 4, 2, 1)}
    pv = wb[pl.ds(_T * _CH * 128, 16)]
    av = jnp.full((16,), 1.0, jnp.float32) * pv[8]
    a = 1.0 / (1.0 + jnp.exp(-av))
    one_m_a = 1.0 - a
    f32 = jnp.float32
    tok0 = wid * _TPW

    def start_in(g, par, sem):
        base = (tok0 + g * _G) * _D
        for t in range(_T):
            pltpu.async_copy(
                x.at[pl.ds(t * _NSC * _D + base, _GB)],
                xb.at[pl.ds((par * _T + t) * _GB, _GB)], sem)

    def drain_in(par, sem):
        for t in range(_T):
            pltpu.make_async_copy(
                x.at[pl.ds(0, _GB)],
                xb.at[pl.ds((par * _T + t) * _GB, _GB)], sem).wait()

    start_in(0, 0, si0)
    start_in(1, 1, si1)

    def run_group(g, par, si, so):
        drain_in(par, si)

        L = [jnp.zeros((16,), f32) for _ in range(8)]
        for blk in range(_G // _NT):
            def jbody(jj, accs):
                accs = list(accs)
                for t in range(_T):
                    wo = t * _CH * 128
                    wA = [wb[pl.ds(wo + (2 * jj) * 128 + k * 16, 16)]
                          for k in range(8)]
                    wB = [wb[pl.ds(wo + (2 * jj + 1) * 128 + k * 16, 16)]
                          for k in range(8)]
                    for n in range(_NT):
                        xo = ((par * _T + t) * _G + blk * _NT + n) * _D
                        r0 = _round_bf16(xb[pl.ds(xo + (2 * jj) * 16, 16)])
                        r1 = _round_bf16(xb[pl.ds(xo + (2 * jj + 1) * 16, 16)])
                        for k in range(8):
                            accs[n * 8 + k] = (accs[n * 8 + k]
                                               + r0 * wA[k] + r1 * wB[k])
                return tuple(accs)

            accs = lax.fori_loop(
                0, _CH // 2, jbody,
                tuple(jnp.zeros((16,), f32) for _ in range(_NT * 8)))
            for n in range(_NT):
                tok = blk * _NT + n
                for k in range(8):
                    s = accs[n * 8 + k]
                    for sh in (8, 4, 2, 1):
                        s = s + _lane_perm(s, perms[sh])
                    L[k] = jnp.where(lanes == tok, s, L[k])

        lt = [L[k] + pv[k] for k in range(4)]
        ls = [L[4 + k] + pv[4 + k] for k in range(4)]
        v1 = jnp.maximum(jnp.maximum(lt[0], lt[1]), jnp.maximum(lt[2], lt[3]))
        i1 = jnp.where(lt[0] >= v1, 0,
                       jnp.where(lt[1] >= v1, 1,
                                 jnp.where(lt[2] >= v1, 2, 3)))
        neg = jnp.float32(-3.0e38)
        l2 = [jnp.where(i1 == k, neg, lt[k]) for k in range(4)]
        v2 = jnp.maximum(jnp.maximum(l2[0], l2[1]), jnp.maximum(l2[2], l2[3]))
        i2 = jnp.where(l2[0] >= v2, 0,
                       jnp.where(l2[1] >= v2, 1,
                                 jnp.where(l2[2] >= v2, 2, 3)))
        e2 = jnp.exp(v2 - v1)
        p1 = 1.0 / (1.0 + e2)
        p2 = 1.0 - p1
        ms = jnp.maximum(jnp.maximum(ls[0], ls[1]), jnp.maximum(ls[2], ls[3]))
        es = [jnp.exp(ls[k] - ms) for k in range(4)]
        den = es[0] + es[1] + es[2] + es[3]
        wts = []
        for k in range(4):
            tw = (jnp.where(i1 == k, p1, 0.0) + jnp.where(i2 == k, p2, 0.0))
            wts.append(a * tw + one_m_a * (es[k] / den))

        obase = par * _GB
        for n in range(_G):
            s0 = wts[0][n]
            s1 = wts[1][n]
            s2 = wts[2][n]
            s3 = wts[3][n]
            xo0 = ((par * _T + 0) * _G + n) * _D
            xo1 = ((par * _T + 1) * _G + n) * _D
            xo2 = ((par * _T + 2) * _G + n) * _D
            xo3 = ((par * _T + 3) * _G + n) * _D

            def cbody(cc, carry2):
                for u in range(4):
                    o = (cc * 4 + u) * 16
                    ob[pl.ds(obase + n * _D + o, 16)] = (
                        xb[pl.ds(xo0 + o, 16)] * s0
                        + xb[pl.ds(xo1 + o, 16)] * s1
                        + xb[pl.ds(xo2 + o, 16)] * s2
                        + xb[pl.ds(xo3 + o, 16)] * s3)
                return carry2

            lax.fori_loop(0, _CH // 4, cbody, 0)

        base = (tok0 + g * _G) * _D
        pltpu.async_copy(ob.at[pl.ds(obase, _GB)],
                         out.at[pl.ds(base, _GB)], so)

    def pair(i, carry):
        for par, si, so in ((0, si0, so0), (1, si1, so1)):
            g = 2 * i + par

            @pl.when(i >= 1)
            def _():
                pltpu.make_async_copy(ob.at[pl.ds(par * _GB, _GB)],
                                      out.at[pl.ds(0, _GB)], so).wait()

            run_group(g, par, si, so)

            @pl.when(i < (_NG // 2) - 1)
            def _():
                start_in(g + 2, par, si)
        return carry

    lax.fori_loop(0, _NG // 2, pair, 0)

    for par, so in ((0, so0), (1, so1)):
        pltpu.make_async_copy(ob.at[pl.ds(par * _GB, _GB)],
                              out.at[pl.ds(0, _GB)], so).wait()


def _build_sc_call():
    mesh = plsc.VectorSubcoreMesh(core_axis_name="c", subcore_axis_name="s")
    return pl.kernel(
        _sc_body,
        mesh=mesh,
        out_type=jax.ShapeDtypeStruct((_NSC * _D,), jnp.float32),
        scratch_types=[
            pltpu.VMEM((2 * _T * _GB,), jnp.float32),
            pltpu.VMEM((2 * _GB,), jnp.float32),
            pltpu.VMEM((_T * _CH * 128 + 16,), jnp.float32),
            pltpu.SemaphoreType.DMA,
            pltpu.SemaphoreType.DMA,
            pltpu.SemaphoreType.DMA,
            pltpu.SemaphoreType.DMA,
        ],
    )


def kernel(mod0, mod1, mod2, mod3, W_top, b_top, W_soft, b_soft, alpha):
    B, S, D = mod0.shape
    N = B * S
    x2d = [m.reshape(N, D) for m in (mod0, mod1, mod2, mod3)]
    xfl = [m.reshape(N, D)[_NTC:].reshape(_NSC * D)
           for m in (mod0, mod1, mod2, mod3)]

    wt = W_top.reshape(_T, D, _T).transpose(2, 1, 0)
    ws = W_soft.reshape(_T, D, _T).transpose(2, 1, 0)
    w = jnp.concatenate([wt, ws], axis=-1)
    bias = jnp.concatenate([b_top, b_soft]).reshape(1, 2 * _T)
    al = alpha.reshape(1, 1)

    wsc = w.astype(jnp.bfloat16).astype(jnp.float32)
    wsc = wsc.reshape(_T, _CH, 16, 8).transpose(0, 1, 3, 2).reshape(-1)
    params = jnp.concatenate(
        [b_top, b_soft, alpha, jnp.zeros((7,), jnp.float32)])
    wp = jnp.concatenate([wsc, params])
    xcat = jnp.concatenate(xfl)

    out_sc = _build_sc_call()(xcat, wp)
    out_tc = _tc_call(x2d, w, bias, al)
    out = jnp.concatenate([out_tc.reshape(-1), out_sc])
    return out.reshape(B, S, D)
